# baseline (speedup 1.0000x reference)
import functools

import jax
import jax.numpy as jnp
from jax import lax
from jax.experimental import pallas as pl
from jax.experimental.pallas import tpu as pltpu
from jax.experimental.pallas import tpu_sc as plsc

DIM = 128
LANES = 16
SLICES = DIM // LANES
CHUNK = 128


@functools.lru_cache(maxsize=None)
def _gather_sumsq(B, num_ent, num_rel):
    info = plsc.get_sparse_core_info()
    NC, NS = info.num_cores, info.num_subcores
    NW = NC * NS
    assert B % NW == 0
    b_per_w = B // NW
    assert b_per_w % CHUNK == 0
    nch = b_per_w // CHUNK

    mesh = plsc.VectorSubcoreMesh(core_axis_name="c", subcore_axis_name="s")

    nch_t = 3 * nch
    NBUF = 4

    @functools.partial(
        pl.kernel,
        mesh=mesh,
        out_type=[
            jax.ShapeDtypeStruct((3 * B, DIM), jnp.float32),
            jax.ShapeDtypeStruct((NW, DIM), jnp.float32),
        ],
        scratch_types=[
            pltpu.VMEM((3 * b_per_w,), jnp.int32),
            pltpu.VMEM((CHUNK, DIM), jnp.float32),
            pltpu.VMEM((CHUNK, DIM), jnp.float32),
            pltpu.VMEM((CHUNK, DIM), jnp.float32),
            pltpu.VMEM((CHUNK, DIM), jnp.float32),
            pltpu.VMEM((DIM,), jnp.float32),
            pltpu.SemaphoreType.DMA,
            pltpu.SemaphoreType.DMA,
            pltpu.SemaphoreType.DMA,
            pltpu.SemaphoreType.DMA,
            pltpu.SemaphoreType.DMA,
            pltpu.SemaphoreType.DMA,
            pltpu.SemaphoreType.DMA,
            pltpu.SemaphoreType.DMA,
            pltpu.SemaphoreType.DMA,
        ],
    )
    def k(ent_hbm, rel_hbm, h_hbm, r_hbm, t_hbm, out_hbm, ss_hbm,
          idx_v, buf0, buf1, buf2, buf3, acc_v,
          g0, g1, g2, g3, w0, w1, w2, w3, isem):
        bufs = [buf0, buf1, buf2, buf3]
        gsems = [g0, g1, g2, g3]
        wsems = [w0, w1, w2, w3]
        wid = lax.axis_index("s") * NC + lax.axis_index("c")
        base = wid * b_per_w
        tabs = [ent_hbm, rel_hbm, ent_hbm]

        idx_cps = [
            pltpu.async_copy(ih.at[pl.ds(base, b_per_w)],
                             idx_v.at[pl.ds(seg * b_per_w, b_per_w)], isem)
            for seg, ih in enumerate([h_hbm, r_hbm, t_hbm])
        ]
        for cp in idx_cps:
            cp.wait()

        def start_gather(c, buf, sem):
            seg = c // nch
            off = seg * b_per_w + (c % nch) * CHUNK
            return pltpu.async_copy(
                tabs[seg].at[idx_v.at[pl.ds(off, CHUNK)]], buf, sem)

        def start_wb(c, buf, sem):
            seg = c // nch
            off = seg * B + base + (c % nch) * CHUNK
            return pltpu.async_copy(buf, out_hbm.at[pl.ds(off, CHUNK)], sem)

        gathers = [None] * nch_t
        wbs = [None] * nch_t
        for c in range(min(NBUF, nch_t)):
            gathers[c] = start_gather(c, bufs[c % NBUF], gsems[c % NBUF])

        acc = tuple(jnp.zeros((LANES,), jnp.float32) for _ in range(SLICES))
        for c in range(nch_t):
            b = c % NBUF
            r0 = c - 1 + NBUF
            if c >= 1 and r0 < nch_t:
                pb = (c - 1) % NBUF
                wbs[c - 1].wait()
                gathers[r0] = start_gather(r0, bufs[pb], gsems[pb])
            gathers[c].wait()
            wbs[c] = start_wb(c, bufs[b], wsems[b])

            def body(i, a, _buf=bufs[b]):
                new = []
                for j in range(SLICES):
                    v = _buf[i, pl.ds(j * LANES, LANES)]
                    new.append(a[j] + v * v)
                return tuple(new)

            acc = lax.fori_loop(0, CHUNK, body, acc, unroll=2)
        for c in range(max(0, nch_t - NBUF), nch_t):
            wbs[c].wait()

        for j in range(SLICES):
            acc_v[pl.ds(j * LANES, LANES)] = acc[j]
        pltpu.sync_copy(acc_v, ss_hbm.at[wid])

    return k


@functools.lru_cache(maxsize=None)
def _normalize(total_rows, nw):
    rows_per_blk = 2048
    assert total_rows % rows_per_blk == 0
    nblk = total_rows // rows_per_blk

    def body(x_ref, ss_ref, o_ref):
        s = jnp.sum(ss_ref[...], axis=0, keepdims=True)
        inv = 1.0 / jnp.maximum(jnp.sqrt(s), 1e-12)
        o_ref[...] = x_ref[...] * inv

    return pl.pallas_call(
        body,
        grid=(nblk,),
        in_specs=[
            pl.BlockSpec((rows_per_blk, DIM), lambda i: (i, 0)),
            pl.BlockSpec((nw, DIM), lambda i: (0, 0)),
        ],
        out_specs=pl.BlockSpec((rows_per_blk, DIM), lambda i: (i, 0)),
        out_shape=jax.ShapeDtypeStruct((total_rows, DIM), jnp.float32),
    )


def kernel(h, r, t, entity_emb, rel_emb):
    B = h.shape[0]
    h = h.astype(jnp.int32)
    r = r.astype(jnp.int32)
    t = t.astype(jnp.int32)
    entity_emb = entity_emb.astype(jnp.float32)
    rel_emb = rel_emb.astype(jnp.float32)
    gathered, ss = _gather_sumsq(B, entity_emb.shape[0], rel_emb.shape[0])(
        entity_emb, rel_emb, h, r, t)
    return _normalize(3 * B, ss.shape[0])(gathered, ss)


# speedup vs baseline: 1.1422x; 1.1422x over previous
import functools

import jax
import jax.numpy as jnp
from jax import lax
from jax.experimental import pallas as pl
from jax.experimental.pallas import tpu as pltpu
from jax.experimental.pallas import tpu_sc as plsc

DIM = 128
LANES = 16
SLICES = DIM // LANES
CHUNK = 128


@functools.lru_cache(maxsize=None)
def _gather_sumsq(B, num_ent, num_rel):
    info = plsc.get_sparse_core_info()
    NC, NS = info.num_cores, info.num_subcores
    NW = NC * NS
    assert B % NW == 0
    b_per_w = B // NW
    assert b_per_w % CHUNK == 0
    nch = b_per_w // CHUNK

    mesh = plsc.VectorSubcoreMesh(core_axis_name="c", subcore_axis_name="s")

    nch_t = 3 * nch
    NBUF = 4

    @functools.partial(
        pl.kernel,
        mesh=mesh,
        out_type=[
            jax.ShapeDtypeStruct((3 * B, DIM), jnp.float32),
            jax.ShapeDtypeStruct((NW, DIM), jnp.float32),
        ],
        scratch_types=[
            pltpu.VMEM((3 * b_per_w,), jnp.int32),
            pltpu.VMEM((CHUNK, DIM), jnp.float32),
            pltpu.VMEM((CHUNK, DIM), jnp.float32),
            pltpu.VMEM((CHUNK, DIM), jnp.float32),
            pltpu.VMEM((CHUNK, DIM), jnp.float32),
            pltpu.VMEM((DIM,), jnp.float32),
            pltpu.SemaphoreType.DMA,
            pltpu.SemaphoreType.DMA,
            pltpu.SemaphoreType.DMA,
            pltpu.SemaphoreType.DMA,
            pltpu.SemaphoreType.DMA,
            pltpu.SemaphoreType.DMA,
            pltpu.SemaphoreType.DMA,
            pltpu.SemaphoreType.DMA,
            pltpu.SemaphoreType.DMA,
        ],
    )
    def k(ent_hbm, rel_hbm, h_hbm, r_hbm, t_hbm, out_hbm, ss_hbm,
          idx_v, buf0, buf1, buf2, buf3, acc_v,
          g0, g1, g2, g3, w0, w1, w2, w3, isem):
        bufs = [buf0, buf1, buf2, buf3]
        gsems = [g0, g1, g2, g3]
        wsems = [w0, w1, w2, w3]
        wid = lax.axis_index("s") * NC + lax.axis_index("c")
        base = wid * b_per_w
        tabs = [ent_hbm, rel_hbm, ent_hbm]

        idx_cps = [
            pltpu.async_copy(ih.at[pl.ds(base, b_per_w)],
                             idx_v.at[pl.ds(seg * b_per_w, b_per_w)], isem)
            for seg, ih in enumerate([h_hbm, r_hbm, t_hbm])
        ]
        for cp in idx_cps:
            cp.wait()

        def start_gather(c, buf, sem):
            seg = c // nch
            off = seg * b_per_w + (c % nch) * CHUNK
            return pltpu.async_copy(
                tabs[seg].at[idx_v.at[pl.ds(off, CHUNK)]], buf, sem)

        def start_wb(c, buf, sem):
            seg = c // nch
            off = seg * B + base + (c % nch) * CHUNK
            return pltpu.async_copy(buf, out_hbm.at[pl.ds(off, CHUNK)], sem)

        gathers = [None] * nch_t
        wbs = [None] * nch_t
        for c in range(min(NBUF, nch_t)):
            gathers[c] = start_gather(c, bufs[c % NBUF], gsems[c % NBUF])

        acc = tuple(jnp.zeros((LANES,), jnp.float32) for _ in range(SLICES))
        for c in range(nch_t):
            b = c % NBUF
            r0 = c - 1 + NBUF
            if c >= 1 and r0 < nch_t:
                pb = (c - 1) % NBUF
                wbs[c - 1].wait()
                gathers[r0] = start_gather(r0, bufs[pb], gsems[pb])
            gathers[c].wait()
            wbs[c] = start_wb(c, bufs[b], wsems[b])

            def body(i, a, _buf=bufs[b]):
                new = []
                for j in range(SLICES):
                    v = _buf[i, pl.ds(j * LANES, LANES)]
                    new.append(a[j] + v * v)
                return tuple(new)

            acc = lax.fori_loop(0, CHUNK, body, acc, unroll=2)
        for c in range(max(0, nch_t - NBUF), nch_t):
            wbs[c].wait()

        for j in range(SLICES):
            acc_v[pl.ds(j * LANES, LANES)] = acc[j]
        pltpu.sync_copy(acc_v, ss_hbm.at[wid])

    return k


@functools.lru_cache(maxsize=None)
def _normalize(total_rows, nw):
    rows_per_blk = 8192
    assert total_rows % rows_per_blk == 0
    nblk = total_rows // rows_per_blk

    def body(x_ref, ss_ref, o_ref):
        s = jnp.sum(ss_ref[...], axis=0, keepdims=True)
        inv = 1.0 / jnp.maximum(jnp.sqrt(s), 1e-12)
        o_ref[...] = x_ref[...] * inv

    return pl.pallas_call(
        body,
        grid=(nblk,),
        in_specs=[
            pl.BlockSpec((rows_per_blk, DIM), lambda i: (i, 0)),
            pl.BlockSpec((nw, DIM), lambda i: (0, 0)),
        ],
        out_specs=pl.BlockSpec((rows_per_blk, DIM), lambda i: (i, 0)),
        out_shape=jax.ShapeDtypeStruct((total_rows, DIM), jnp.float32),
    )


def kernel(h, r, t, entity_emb, rel_emb):
    B = h.shape[0]
    h = h.astype(jnp.int32)
    r = r.astype(jnp.int32)
    t = t.astype(jnp.int32)
    entity_emb = entity_emb.astype(jnp.float32)
    rel_emb = rel_emb.astype(jnp.float32)
    gathered, ss = _gather_sumsq(B, entity_emb.shape[0], rel_emb.shape[0])(
        entity_emb, rel_emb, h, r, t)
    return _normalize(3 * B, ss.shape[0])(gathered, ss)


# speedup vs baseline: 1.1683x; 1.0228x over previous
import functools

import jax
import jax.numpy as jnp
from jax import lax
from jax.experimental import pallas as pl
from jax.experimental.pallas import tpu as pltpu
from jax.experimental.pallas import tpu_sc as plsc

DIM = 128
LANES = 16
SLICES = DIM // LANES
CHUNK = 128


@functools.lru_cache(maxsize=None)
def _gather_sumsq(B, num_ent, num_rel):
    info = plsc.get_sparse_core_info()
    NC, NS = info.num_cores, info.num_subcores
    NW = NC * NS
    assert B % NW == 0
    b_per_w = B // NW
    assert b_per_w % CHUNK == 0
    nch = b_per_w // CHUNK

    mesh = plsc.VectorSubcoreMesh(core_axis_name="c", subcore_axis_name="s")

    nch_t = 3 * nch
    NBUF = 4

    @functools.partial(
        pl.kernel,
        mesh=mesh,
        out_type=[
            jax.ShapeDtypeStruct((3 * B, DIM), jnp.float32),
            jax.ShapeDtypeStruct((NW, DIM), jnp.float32),
        ],
        scratch_types=[
            pltpu.VMEM((3 * b_per_w,), jnp.int32),
            pltpu.VMEM((CHUNK, DIM), jnp.float32),
            pltpu.VMEM((CHUNK, DIM), jnp.float32),
            pltpu.VMEM((CHUNK, DIM), jnp.float32),
            pltpu.VMEM((CHUNK, DIM), jnp.float32),
            pltpu.VMEM((DIM,), jnp.float32),
            pltpu.SemaphoreType.DMA,
            pltpu.SemaphoreType.DMA,
            pltpu.SemaphoreType.DMA,
            pltpu.SemaphoreType.DMA,
            pltpu.SemaphoreType.DMA,
            pltpu.SemaphoreType.DMA,
            pltpu.SemaphoreType.DMA,
            pltpu.SemaphoreType.DMA,
            pltpu.SemaphoreType.DMA,
        ],
    )
    def k(ent_hbm, rel_hbm, h_hbm, r_hbm, t_hbm, out_hbm, ss_hbm,
          idx_v, buf0, buf1, buf2, buf3, acc_v,
          g0, g1, g2, g3, w0, w1, w2, w3, isem):
        bufs = [buf0, buf1, buf2, buf3]
        gsems = [g0, g1, g2, g3]
        wsems = [w0, w1, w2, w3]
        wid = lax.axis_index("s") * NC + lax.axis_index("c")
        base = wid * b_per_w
        tabs = [ent_hbm, rel_hbm, ent_hbm]

        idx_cps = [
            pltpu.async_copy(ih.at[pl.ds(base, b_per_w)],
                             idx_v.at[pl.ds(seg * b_per_w, b_per_w)], isem)
            for seg, ih in enumerate([h_hbm, r_hbm, t_hbm])
        ]
        for cp in idx_cps:
            cp.wait()

        def start_gather(c, buf, sem):
            seg = c // nch
            off = seg * b_per_w + (c % nch) * CHUNK
            return pltpu.async_copy(
                tabs[seg].at[idx_v.at[pl.ds(off, CHUNK)]], buf, sem)

        def start_wb(c, buf, sem):
            seg = c // nch
            off = seg * B + base + (c % nch) * CHUNK
            return pltpu.async_copy(buf, out_hbm.at[pl.ds(off, CHUNK)], sem)

        gathers = [None] * nch_t
        wbs = [None] * nch_t
        for c in range(min(NBUF, nch_t)):
            gathers[c] = start_gather(c, bufs[c % NBUF], gsems[c % NBUF])

        acc = tuple(jnp.zeros((LANES,), jnp.float32) for _ in range(SLICES))
        for c in range(nch_t):
            b = c % NBUF
            r0 = c - 1 + NBUF
            if c >= 1 and r0 < nch_t:
                pb = (c - 1) % NBUF
                wbs[c - 1].wait()
                gathers[r0] = start_gather(r0, bufs[pb], gsems[pb])
            gathers[c].wait()
            wbs[c] = start_wb(c, bufs[b], wsems[b])

            def body(i, a, _buf=bufs[b]):
                new = []
                for j in range(SLICES):
                    v = _buf[i, pl.ds(j * LANES, LANES)]
                    new.append(a[j] + v * v)
                return tuple(new)

            acc = lax.fori_loop(0, CHUNK, body, acc, unroll=2)
        for c in range(max(0, nch_t - NBUF), nch_t):
            wbs[c].wait()

        for j in range(SLICES):
            acc_v[pl.ds(j * LANES, LANES)] = acc[j]
        pltpu.sync_copy(acc_v, ss_hbm.at[wid])

    return k


@functools.lru_cache(maxsize=None)
def _normalize(total_rows, nw):
    rows_per_blk = 16384
    assert total_rows % rows_per_blk == 0
    nblk = total_rows // rows_per_blk

    def body(x_ref, ss_ref, o_ref):
        s = jnp.sum(ss_ref[...], axis=0, keepdims=True)
        inv = 1.0 / jnp.maximum(jnp.sqrt(s), 1e-12)
        o_ref[...] = x_ref[...] * inv

    return pl.pallas_call(
        body,
        grid=(nblk,),
        in_specs=[
            pl.BlockSpec((rows_per_blk, DIM), lambda i: (i, 0)),
            pl.BlockSpec((nw, DIM), lambda i: (0, 0)),
        ],
        out_specs=pl.BlockSpec((rows_per_blk, DIM), lambda i: (i, 0)),
        out_shape=jax.ShapeDtypeStruct((total_rows, DIM), jnp.float32),
    )


def kernel(h, r, t, entity_emb, rel_emb):
    B = h.shape[0]
    h = h.astype(jnp.int32)
    r = r.astype(jnp.int32)
    t = t.astype(jnp.int32)
    entity_emb = entity_emb.astype(jnp.float32)
    rel_emb = rel_emb.astype(jnp.float32)
    gathered, ss = _gather_sumsq(B, entity_emb.shape[0], rel_emb.shape[0])(
        entity_emb, rel_emb, h, r, t)
    return _normalize(3 * B, ss.shape[0])(gathered, ss)


# speedup vs baseline: 1.1784x; 1.0086x over previous
import functools

import jax
import jax.numpy as jnp
from jax import lax
from jax.experimental import pallas as pl
from jax.experimental.pallas import tpu as pltpu
from jax.experimental.pallas import tpu_sc as plsc

DIM = 128
LANES = 16
SLICES = DIM // LANES
CHUNK = 128


@functools.lru_cache(maxsize=None)
def _gather_sumsq(B, num_ent, num_rel):
    info = plsc.get_sparse_core_info()
    NC, NS = info.num_cores, info.num_subcores
    NW = NC * NS
    assert B % NW == 0
    b_per_w = B // NW
    assert b_per_w % CHUNK == 0
    nch = b_per_w // CHUNK

    mesh = plsc.VectorSubcoreMesh(core_axis_name="c", subcore_axis_name="s")

    nch_t = 3 * nch
    NBUF = 4

    @functools.partial(
        pl.kernel,
        mesh=mesh,
        out_type=[
            jax.ShapeDtypeStruct((3 * B, DIM), jnp.float32),
            jax.ShapeDtypeStruct((NW, DIM), jnp.float32),
        ],
        scratch_types=[
            pltpu.VMEM((3 * b_per_w,), jnp.int32),
            pltpu.VMEM((CHUNK, DIM), jnp.float32),
            pltpu.VMEM((CHUNK, DIM), jnp.float32),
            pltpu.VMEM((CHUNK, DIM), jnp.float32),
            pltpu.VMEM((CHUNK, DIM), jnp.float32),
            pltpu.VMEM((DIM,), jnp.float32),
            pltpu.SemaphoreType.DMA,
            pltpu.SemaphoreType.DMA,
            pltpu.SemaphoreType.DMA,
            pltpu.SemaphoreType.DMA,
            pltpu.SemaphoreType.DMA,
            pltpu.SemaphoreType.DMA,
            pltpu.SemaphoreType.DMA,
            pltpu.SemaphoreType.DMA,
            pltpu.SemaphoreType.DMA,
        ],
    )
    def k(ent_hbm, rel_hbm, h_hbm, r_hbm, t_hbm, out_hbm, ss_hbm,
          idx_v, buf0, buf1, buf2, buf3, acc_v,
          g0, g1, g2, g3, w0, w1, w2, w3, isem):
        bufs = [buf0, buf1, buf2, buf3]
        gsems = [g0, g1, g2, g3]
        wsems = [w0, w1, w2, w3]
        wid = lax.axis_index("s") * NC + lax.axis_index("c")
        base = wid * b_per_w
        tabs = [ent_hbm, rel_hbm, ent_hbm]

        idx_cps = [
            pltpu.async_copy(ih.at[pl.ds(base, b_per_w)],
                             idx_v.at[pl.ds(seg * b_per_w, b_per_w)], isem)
            for seg, ih in enumerate([h_hbm, r_hbm, t_hbm])
        ]
        for cp in idx_cps:
            cp.wait()

        def start_gather(c, buf, sem):
            seg = c // nch
            off = seg * b_per_w + (c % nch) * CHUNK
            return pltpu.async_copy(
                tabs[seg].at[idx_v.at[pl.ds(off, CHUNK)]], buf, sem)

        def start_wb(c, buf, sem):
            seg = c // nch
            off = seg * B + base + (c % nch) * CHUNK
            return pltpu.async_copy(buf, out_hbm.at[pl.ds(off, CHUNK)], sem)

        gathers = [None] * nch_t
        wbs = [None] * nch_t
        for c in range(min(NBUF, nch_t)):
            gathers[c] = start_gather(c, bufs[c % NBUF], gsems[c % NBUF])

        acc = tuple(jnp.zeros((LANES,), jnp.float32) for _ in range(SLICES))
        for c in range(nch_t):
            b = c % NBUF
            r0 = c - 1 + NBUF
            if c >= 1 and r0 < nch_t:
                pb = (c - 1) % NBUF
                wbs[c - 1].wait()
                gathers[r0] = start_gather(r0, bufs[pb], gsems[pb])
            gathers[c].wait()
            wbs[c] = start_wb(c, bufs[b], wsems[b])

            def body(i, a, _buf=bufs[b]):
                new = []
                for j in range(SLICES):
                    v = _buf[i, pl.ds(j * LANES, LANES)]
                    new.append(a[j] + v * v)
                return tuple(new)

            acc = lax.fori_loop(0, CHUNK, body, acc, unroll=2)
        for c in range(max(0, nch_t - NBUF), nch_t):
            wbs[c].wait()

        for j in range(SLICES):
            acc_v[pl.ds(j * LANES, LANES)] = acc[j]
        pltpu.sync_copy(acc_v, ss_hbm.at[wid])

    return k


@functools.lru_cache(maxsize=None)
def _normalize(total_rows, nw):
    rows_per_blk = 24576
    assert total_rows % rows_per_blk == 0
    nblk = total_rows // rows_per_blk

    def body(x_ref, ss_ref, o_ref):
        s = jnp.sum(ss_ref[...], axis=0, keepdims=True)
        inv = 1.0 / jnp.maximum(jnp.sqrt(s), 1e-12)
        o_ref[...] = x_ref[...] * inv

    return pl.pallas_call(
        body,
        grid=(nblk,),
        in_specs=[
            pl.BlockSpec((rows_per_blk, DIM), lambda i: (i, 0)),
            pl.BlockSpec((nw, DIM), lambda i: (0, 0)),
        ],
        out_specs=pl.BlockSpec((rows_per_blk, DIM), lambda i: (i, 0)),
        out_shape=jax.ShapeDtypeStruct((total_rows, DIM), jnp.float32),
    )


def kernel(h, r, t, entity_emb, rel_emb):
    B = h.shape[0]
    h = h.astype(jnp.int32)
    r = r.astype(jnp.int32)
    t = t.astype(jnp.int32)
    entity_emb = entity_emb.astype(jnp.float32)
    rel_emb = rel_emb.astype(jnp.float32)
    gathered, ss = _gather_sumsq(B, entity_emb.shape[0], rel_emb.shape[0])(
        entity_emb, rel_emb, h, r, t)
    return _normalize(3 * B, ss.shape[0])(gathered, ss)
